# speedup vs baseline: 16.7491x; 16.7491x over previous
import jax
import jax.numpy as jnp
from jax import lax
from jax.experimental import pallas as pl
from jax.experimental.pallas import tpu as pltpu
from jax.experimental.pallas import tpu_sc as plsc

D = 128
B = 4096
L = 200
LANES = 16
NVEC = D // LANES

CH0 = 128
CH1 = L - CH0
NBUF = 4

NC = 2
NS = 16
NW = NC * NS
BPW = B // NW
GROUPS = BPW // 2


def _chunk(u):
    return u // 2, (u % 2) * CH0, CH0 if u % 2 == 0 else CH1


def _sc_body(ids_hbm, table_hbm, out_hbm, idx_v, rows_v, out_v,
             sem0, sem1, sem2, sem3):
    sems = (sem0, sem1, sem2, sem3)
    wid = lax.axis_index("s") * NC + lax.axis_index("c")

    pltpu.sync_copy(ids_hbm.at[pl.ds(wid * BPW * L, BPW * L)], idx_v)

    def start(g, u):
        dr, off, sz = _chunk(u)
        row = 2 * g + dr
        pltpu.async_copy(
            table_hbm.at[idx_v.at[pl.ds(row * L + off, sz)]],
            rows_v.at[u, pl.ds(0, sz)],
            sems[u])

    def wait(u):
        _, _, sz = _chunk(u)
        pltpu.make_async_copy(
            table_hbm.at[pl.ds(0, sz)], rows_v.at[u, pl.ds(0, sz)],
            sems[u]).wait()

    for u in range(NBUF):
        start(0, u)

    zeros = (jnp.zeros((LANES,), jnp.float32),) * NVEC

    def accum(u, acc):
        _, _, sz = _chunk(u)
        rv = rows_v.at[u]

        def jbody(j, a):
            return tuple(a[k] + rv[j, pl.ds(k * LANES, LANES)]
                         for k in range(NVEC))

        return lax.fori_loop(0, sz, jbody, acc)

    def gbody(g, carry):
        acc = zeros
        for u in range(NBUF):
            wait(u)

            @pl.when(g < GROUPS - 1)
            def _():
                start(g + 1, u)

            if u % 2 == 0:
                acc = accum(u, zeros)
            else:
                acc = accum(u, acc)
                row = 2 * g + u // 2
                for k in range(NVEC):
                    out_v[row, pl.ds(k * LANES, LANES)] = acc[k]
        return carry

    lax.fori_loop(0, GROUPS, gbody, 0)
    pltpu.sync_copy(out_v, out_hbm.at[pl.ds(wid * BPW, BPW)])


def kernel(input_ids, table):
    ids_flat = input_ids.reshape(B * L).astype(jnp.int32)
    f = pl.kernel(
        _sc_body,
        mesh=plsc.VectorSubcoreMesh(core_axis_name="c", subcore_axis_name="s"),
        out_type=jax.ShapeDtypeStruct((B, D), jnp.float32),
        scratch_types=[
            pltpu.VMEM((BPW * L,), jnp.int32),
            pltpu.VMEM((NBUF, CH0, D), jnp.float32),
            pltpu.VMEM((BPW, D), jnp.float32),
            pltpu.SemaphoreType.DMA,
            pltpu.SemaphoreType.DMA,
            pltpu.SemaphoreType.DMA,
            pltpu.SemaphoreType.DMA,
        ],
    )
    return f(ids_flat, table)
